# Initial kernel scaffold; baseline (speedup 1.0000x reference)
import jax
import jax.numpy as jnp
from jax.experimental import pallas as pl


def kernel(inputs, neighList, params):
    raise NotImplementedError("write your pallas kernel here")


# speedup vs baseline: 1.9059x; 1.9059x over previous
import functools

import jax
import jax.numpy as jnp
import numpy as np
from jax import lax
from jax.experimental import pallas as pl
from jax.experimental.pallas import tpu as pltpu
from jax.experimental.pallas import tpu_sc as plsc

B = 2
NP = 10000
MNN = 32
LBOX = 10.0
KF = 500
KP = 512
FFTC = 4
P = NP * MNN
BP = B * P
RT = 1000
TP = RT * MNN
NT = 10
RTP = 80
TPP = RTP * MNN
NTP = NP // RTP
NW = 32
PPW = BP // NW

_KV = (2.0 * np.pi / LBOX) * np.arange(-(KF // 2), KF // 2, dtype=np.float32)
KPAD = np.zeros((1, KP), dtype=np.float32)
KPAD[0, :KF] = _KV

_BF = jnp.bfloat16


def _q(a):
    return a.astype(_BF)


def _dotq(a, b):
    return jnp.dot(_q(a), _q(b), preferred_element_type=jnp.float32)


def _full_spec(shape):
    nd = len(shape)
    return pl.BlockSpec(shape, lambda b, t: (0,) * nd)


def _wb(wrefs, i):
    return wrefs[2 * i][...], wrefs[2 * i + 1][...]


def _pair_mlp_fwd(g, wrefs):
    x = g
    ys = []
    for i in range(5):
        W, b = _wb(wrefs, i)
        if i == 0:
            pre = x * W[0][None, :] + b
        else:
            pre = _dotq(x, W) + b
        y = jnp.tanh(pre)
        ys.append(y)
        x = y + jnp.concatenate([x, x], axis=-1)
    return x, ys


def _pair_mlp_bwd(ys, wrefs, dx):
    for i in reversed(range(5)):
        W, _ = _wb(wrefs, i)
        y = ys[i]
        dpre = dx * (1.0 - y * y)
        din = W.shape[0]
        dx = _dotq(dpre, W.T) + dx[:, :din] + dx[:, din:]
    return dx


def _pair_geom(d, vd):
    dist = jnp.abs(d)
    ok = vd > 0.0
    safe = jnp.where(ok, jnp.maximum(dist, 1e-6), 1.0)
    g0 = jnp.where(ok, 1.0 / safe, 0.0)
    g1 = jnp.where(ok, safe, 0.0)
    return dist, ok, safe, g0, g1


def _rho_body(x_ref, k_ref, dep_ref, rho_ref):
    del dep_ref
    ph = x_ref[0] * k_ref[...]
    c = jnp.cos(ph)
    s = jnp.sin(ph)
    blk = jnp.concatenate(
        [jnp.sum(c, axis=0, keepdims=True), -jnp.sum(s, axis=0, keepdims=True)],
        axis=0)

    @pl.when(pl.program_id(1) == 0)
    def _():
        rho_ref[0] = blk

    @pl.when(pl.program_id(1) != 0)
    def _():
        rho_ref[0] = rho_ref[0] + blk


def _pairfwd_body(diff_ref, valid_ref, *refs):
    wrefs, d_out = refs[:20], refs[20]
    d = diff_ref[0]
    vd = valid_ref[0]
    _, _, _, g0, g1 = _pair_geom(d, vd)
    p1, _ = _pair_mlp_fwd(g1, wrefs[:10])
    p2, _ = _pair_mlp_fwd(g0, wrefs[10:])
    ll = jnp.concatenate([p1 * g0, p2 * g0], axis=-1)
    d_out[0] = jnp.sum(ll.reshape(RTP, MNN, 64), axis=1)


def _lr_body(x_ref, k_ref, rho_ref, mult_ref, lr_ref):
    ph = x_ref[0] * k_ref[...]
    c = jnp.cos(ph)
    s = jnp.sin(ph)
    are = mult_ref[...] * rho_ref[0, 0:1, :]
    aim = mult_ref[...] * rho_ref[0, 1:2, :]
    f = (jnp.einsum('tk,ck->tc', _q(c), _q(are),
                    preferred_element_type=jnp.float32)
         - jnp.einsum('tk,ck->tc', _q(s), _q(aim),
                      preferred_element_type=jnp.float32)) / NP
    lr_ref[0] = f


def _head_body(d_ref, lr_ref, *refs):
    wlr = refs[:10]
    wfit = refs[10:20]
    linw_ref, linb_ref, e_ref, dd_ref, dlr_ref = refs[20:]
    lacts = []
    x = lr_ref[0]
    for i in range(5):
        W, b = _wb(wlr, i)
        pre = _dotq(x, W) + b
        y = jnp.maximum(pre, 0.0)
        lacts.append(y)
        x = y if i == 0 else y + jnp.concatenate([x, x], axis=-1)
    dlong = jnp.concatenate([d_ref[0], x], axis=-1)
    facts = []
    x = dlong
    for i in range(5):
        W, b = _wb(wfit, i)
        y = jnp.tanh(_dotq(x, W) + b)
        facts.append((y, W))
        x = y
    lw = linw_ref[...]
    f = x * lw[0][None, :] + linb_ref[...]
    esum = jnp.sum(f).reshape(1, 1, 1)

    @pl.when(pl.program_id(1) == 0)
    def _():
        e_ref[...] = esum

    @pl.when(pl.program_id(1) != 0)
    def _():
        e_ref[...] = e_ref[...] + esum

    dx = jnp.broadcast_to(lw[0:1, 0:1], (RT, 1))
    for (y, W) in reversed(facts):
        dpre = dx * (1.0 - y * y)
        if W.shape[1] == 1:
            dx = dpre * W[:, 0][None, :]
        else:
            dx = _dotq(dpre, W.T)
    dd_ref[0] = dx[:, :64]
    dx = dx[:, 64:]
    for i in reversed(range(5)):
        W, _ = _wb(wlr, i)
        y = lacts[i]
        dpre = dx * (y > 0.0)
        dxin = _dotq(dpre, W.T)
        if i > 0:
            din = W.shape[0]
            dxin = dxin + dx[:, :din] + dx[:, din:]
        dx = dxin
    dlr_ref[0] = dx


def _pairbwd_body(diff_ref, valid_ref, dd_ref, *refs):
    wrefs, gd_ref, dxc_ref = refs[:20], refs[20], refs[21]
    d = diff_ref[0]
    vd = valid_ref[0]
    dist, ok, safe, g0, g1 = _pair_geom(d, vd)
    p1, ys1 = _pair_mlp_fwd(g1, wrefs[:10])
    p2, ys2 = _pair_mlp_fwd(g0, wrefs[10:])
    dd = dd_ref[0]
    up = jnp.broadcast_to(dd[:, None, :32], (RTP, MNN, 32)).reshape(TPP, 32)
    vp = jnp.broadcast_to(dd[:, None, 32:], (RTP, MNN, 32)).reshape(TPP, 32)
    dg1 = _pair_mlp_bwd(ys1, wrefs[:10], up * g0)
    dg0 = (jnp.sum(up * p1, axis=1, keepdims=True)
           + jnp.sum(vp * p2, axis=1, keepdims=True)
           + _pair_mlp_bwd(ys2, wrefs[10:], vp * g0))
    ddist = jnp.where(ok & (dist > 1e-6), dg1 - dg0 / (safe * safe), 0.0)
    gd = ddist * jnp.sign(d)
    gd_ref[0] = gd
    dxc_ref[0] = -jnp.sum(gd.reshape(RTP, MNN, 1), axis=1)


def _nufftbwd1_body(x_ref, dlr_ref, mult_ref, rho_ref, k_ref, dep_ref,
                    da_ref, f1_ref):
    del dep_ref
    ph = x_ref[0] * k_ref[...]
    c = jnp.cos(ph)
    s = jnp.sin(ph)
    gn = dlr_ref[0] * (1.0 / NP)
    gq = _q(gn)
    are = mult_ref[...] * rho_ref[0, 0:1, :]
    aim = mult_ref[...] * rho_ref[0, 1:2, :]
    dc_f = jnp.einsum('tc,ck->tk', gq, _q(are),
                      preferred_element_type=jnp.float32)
    ds_f = -jnp.einsum('tc,ck->tk', gq, _q(aim),
                       preferred_element_type=jnp.float32)
    kv = k_ref[...]
    f1_ref[0] = jnp.sum((-s * dc_f + c * ds_f) * kv, axis=1, keepdims=True)
    dat = jnp.einsum('tc,tk->ck', gq, _q(c), preferred_element_type=jnp.float32)
    dbt = -jnp.einsum('tc,tk->ck', gq, _q(s), preferred_element_type=jnp.float32)
    blk = jnp.concatenate([dat, dbt], axis=0)

    @pl.when(pl.program_id(1) == 0)
    def _():
        da_ref[0] = blk

    @pl.when(pl.program_id(1) != 0)
    def _():
        da_ref[0] = da_ref[0] + blk


def _combine_body(part_ref, dxn_ref):
    dxn_ref[0] = jnp.sum(part_ref[...], axis=0, keepdims=True)


def _force_body(x_ref, da_ref, mult_ref, k_ref, f1_ref, dxc_ref, dxn_ref,
                out_ref):
    ph = x_ref[0] * k_ref[...]
    c = jnp.cos(ph)
    s = jnp.sin(ph)
    kv = k_ref[...]
    m = mult_ref[...]
    drr = jnp.sum(da_ref[0, :FFTC, :] * m, axis=0, keepdims=True)
    dri = jnp.sum(da_ref[0, FFTC:, :] * m, axis=0, keepdims=True)
    t2 = jnp.sum((-s * drr - c * dri) * kv, axis=1, keepdims=True)
    out_ref[0] = -(f1_ref[0] + t2 + dxc_ref[0] + dxn_ref[0])


def _sc_gather(x, neigh_flat):
    mesh = plsc.VectorSubcoreMesh(core_axis_name="c", subcore_axis_name="s")

    @functools.partial(
        pl.kernel, mesh=mesh,
        out_type=[jax.ShapeDtypeStruct((BP,), jnp.float32),
                  jax.ShapeDtypeStruct((BP,), jnp.float32)],
        scratch_types=[pltpu.VMEM((NP,), jnp.float32),
                       pltpu.VMEM((PPW,), jnp.int32),
                       pltpu.VMEM((PPW,), jnp.float32),
                       pltpu.VMEM((PPW,), jnp.float32)],
        compiler_params=pltpu.CompilerParams(needs_layout_passes=False,
                                             has_side_effects=True),
    )
    def k(x_hbm, n_hbm, diff_hbm, valid_hbm, xtab, nbuf, dbuf, vbuf):
        wid = lax.axis_index("s") * 2 + lax.axis_index("c")
        b = wid // 16
        base = wid * PPW
        pltpu.sync_copy(x_hbm.at[b], xtab)
        pltpu.sync_copy(n_hbm.at[pl.ds(base, PPW)], nbuf)
        woff = (wid % 16) * PPW

        def body(j, _):
            o = j * 16
            nv = nbuf[pl.ds(o, 16)]
            pb = woff + o + lax.broadcasted_iota(jnp.int32, (16,), 0)
            iv = lax.shift_right_logical(pb, 5)
            xj = plsc.load_gather(xtab, [nv])
            xi = plsc.load_gather(xtab, [iv])
            d0 = xj - xi
            tt = d0 * jnp.float32(1.0 / LBOX)
            wrap = jnp.where(jnp.abs(tt) > 0.5, jnp.sign(tt), 0.0)
            dbuf[pl.ds(o, 16)] = d0 - jnp.float32(LBOX) * wrap
            okv = (nv >= 0) & (nv != iv)
            vbuf[pl.ds(o, 16)] = jnp.where(okv, 1.0, 0.0).astype(jnp.float32)
            return 0

        lax.fori_loop(0, PPW // 16, body, 0)
        pltpu.sync_copy(dbuf, diff_hbm.at[pl.ds(base, PPW)])
        pltpu.sync_copy(vbuf, valid_hbm.at[pl.ds(base, PPW)])

    return k(x, neigh_flat)


def _sc_scatter(gd_flat, neigh_flat):
    mesh = plsc.VectorSubcoreMesh(core_axis_name="c", subcore_axis_name="s")

    @functools.partial(
        pl.kernel, mesh=mesh,
        out_type=jax.ShapeDtypeStruct((NW, NP), jnp.float32),
        scratch_types=[pltpu.VMEM((NP,), jnp.float32),
                       pltpu.VMEM((PPW,), jnp.int32),
                       pltpu.VMEM((PPW,), jnp.float32)],
        compiler_params=pltpu.CompilerParams(needs_layout_passes=False,
                                             has_side_effects=True),
    )
    def k(g_hbm, n_hbm, part_hbm, acc, nbuf, gbuf):
        wid = lax.axis_index("s") * 2 + lax.axis_index("c")
        base = wid * PPW
        pltpu.sync_copy(n_hbm.at[pl.ds(base, PPW)], nbuf)
        pltpu.sync_copy(g_hbm.at[pl.ds(base, PPW)], gbuf)

        def zbody(j, _):
            acc[pl.ds(j * 16, 16)] = jnp.zeros((16,), jnp.float32)
            return 0

        lax.fori_loop(0, NP // 16, zbody, 0)

        def body(j, _):
            o = j * 16
            plsc.addupdate_scatter(acc, [nbuf[pl.ds(o, 16)]], gbuf[pl.ds(o, 16)])
            return 0

        lax.fori_loop(0, PPW // 16, body, 0)
        pltpu.sync_copy(acc, part_hbm.at[wid])

    return k(gd_flat, neigh_flat)


def kernel(inputs, neighList, params):
    x3 = inputs[:, :, None]
    neigh_flat = neighList.reshape(BP)
    kpad = jnp.asarray(KPAD)
    multp = jnp.concatenate(
        [params['mult'], jnp.zeros((FFTC, KP - KF), jnp.float32)], axis=1)

    def wlist(ps):
        out = []
        for (W, b) in ps:
            out.append(W)
            out.append(b.reshape(1, -1))
        return out

    pyr_w = wlist(params['pyr'])
    pyrinv_w = wlist(params['pyrInv'])
    pyrlr_w = wlist(params['pyrLR'])
    fit_w = wlist(params['fit'])
    linw, linb = params['lin']
    linb = linb.reshape(1, 1)

    w_specs = [_full_spec(w.shape) for w in pyr_w + pyrinv_w]
    grid = (B, NT)

    diff_flat, valid_flat = _sc_gather(inputs, neigh_flat)
    diff3 = diff_flat.reshape(B, P, 1)
    valid3 = valid_flat.reshape(B, P, 1)

    rho = pl.pallas_call(
        _rho_body, grid=grid,
        in_specs=[pl.BlockSpec((1, RT, 1), lambda b, t: (b, t, 0)),
                  _full_spec((1, KP)),
                  pl.BlockSpec((1, 8, 1), lambda b, t: (0, 0, 0))],
        out_specs=pl.BlockSpec((1, 2, KP), lambda b, t: (b, 0, 0)),
        out_shape=jax.ShapeDtypeStruct((B, 2, KP), jnp.float32),
    )(x3, kpad, diff3)

    D = pl.pallas_call(
        _pairfwd_body, grid=(B, NTP),
        in_specs=[pl.BlockSpec((1, TPP, 1), lambda b, t: (b, t, 0)),
                  pl.BlockSpec((1, TPP, 1), lambda b, t: (b, t, 0))] + w_specs,
        out_specs=pl.BlockSpec((1, RTP, 64), lambda b, t: (b, t, 0)),
        out_shape=jax.ShapeDtypeStruct((B, NP, 64), jnp.float32),
    )(diff3, valid3, *pyr_w, *pyrinv_w)

    lr = pl.pallas_call(
        _lr_body, grid=grid,
        in_specs=[pl.BlockSpec((1, RT, 1), lambda b, t: (b, t, 0)),
                  _full_spec((1, KP)),
                  pl.BlockSpec((1, 2, KP), lambda b, t: (b, 0, 0)),
                  _full_spec((FFTC, KP))],
        out_specs=pl.BlockSpec((1, RT, FFTC), lambda b, t: (b, t, 0)),
        out_shape=jax.ShapeDtypeStruct((B, NP, FFTC), jnp.float32),
    )(x3, kpad, rho, multp)

    head_w = pyrlr_w + fit_w
    energy, dD, dlr = pl.pallas_call(
        _head_body, grid=grid,
        in_specs=[pl.BlockSpec((1, RT, 64), lambda b, t: (b, t, 0)),
                  pl.BlockSpec((1, RT, FFTC), lambda b, t: (b, t, 0))]
                 + [_full_spec(w.shape) for w in head_w]
                 + [_full_spec((1, 1)), _full_spec((1, 1))],
        out_specs=[pl.BlockSpec((1, 1, 1), lambda b, t: (b, 0, 0)),
                   pl.BlockSpec((1, RT, 64), lambda b, t: (b, t, 0)),
                   pl.BlockSpec((1, RT, FFTC), lambda b, t: (b, t, 0))],
        out_shape=[jax.ShapeDtypeStruct((B, 1, 1), jnp.float32),
                   jax.ShapeDtypeStruct((B, NP, 64), jnp.float32),
                   jax.ShapeDtypeStruct((B, NP, FFTC), jnp.float32)],
    )(D, lr, *head_w, linw, linb)

    gd3, dxc = pl.pallas_call(
        _pairbwd_body, grid=(B, NTP),
        in_specs=[pl.BlockSpec((1, TPP, 1), lambda b, t: (b, t, 0)),
                  pl.BlockSpec((1, TPP, 1), lambda b, t: (b, t, 0)),
                  pl.BlockSpec((1, RTP, 64), lambda b, t: (b, t, 0))] + w_specs,
        out_specs=[pl.BlockSpec((1, TPP, 1), lambda b, t: (b, t, 0)),
                   pl.BlockSpec((1, RTP, 1), lambda b, t: (b, t, 0))],
        out_shape=[jax.ShapeDtypeStruct((B, P, 1), jnp.float32),
                   jax.ShapeDtypeStruct((B, NP, 1), jnp.float32)],
    )(diff3, valid3, dD, *pyr_w, *pyrinv_w)

    part = _sc_scatter(gd3.reshape(BP), neigh_flat)

    dxn = pl.pallas_call(
        _combine_body, grid=(B,),
        in_specs=[pl.BlockSpec((16, NP), lambda b: (b, 0))],
        out_specs=pl.BlockSpec((1, 1, NP), lambda b: (b, 0, 0)),
        out_shape=jax.ShapeDtypeStruct((B, 1, NP), jnp.float32),
    )(part)

    da, f1 = pl.pallas_call(
        _nufftbwd1_body, grid=grid,
        in_specs=[pl.BlockSpec((1, RT, 1), lambda b, t: (b, t, 0)),
                  pl.BlockSpec((1, RT, FFTC), lambda b, t: (b, t, 0)),
                  _full_spec((FFTC, KP)),
                  pl.BlockSpec((1, 2, KP), lambda b, t: (b, 0, 0)),
                  _full_spec((1, KP)),
                  pl.BlockSpec((1, 1, NP), lambda b, t: (0, 0, 0))],
        out_specs=[pl.BlockSpec((1, 2 * FFTC, KP), lambda b, t: (b, 0, 0)),
                   pl.BlockSpec((1, RT, 1), lambda b, t: (b, t, 0))],
        out_shape=[jax.ShapeDtypeStruct((B, 2 * FFTC, KP), jnp.float32),
                   jax.ShapeDtypeStruct((B, NP, 1), jnp.float32)],
    )(x3, dlr, multp, rho, kpad, dxn)

    forces3 = pl.pallas_call(
        _force_body, grid=grid,
        in_specs=[pl.BlockSpec((1, RT, 1), lambda b, t: (b, t, 0)),
                  pl.BlockSpec((1, 2 * FFTC, KP), lambda b, t: (b, 0, 0)),
                  _full_spec((FFTC, KP)),
                  _full_spec((1, KP)),
                  pl.BlockSpec((1, RT, 1), lambda b, t: (b, t, 0)),
                  pl.BlockSpec((1, RT, 1), lambda b, t: (b, t, 0)),
                  pl.BlockSpec((1, RT, 1), lambda b, t: (b, t, 0))],
        out_specs=pl.BlockSpec((1, RT, 1), lambda b, t: (b, t, 0)),
        out_shape=jax.ShapeDtypeStruct((B, NP, 1), jnp.float32),
    )(x3, da, multp, kpad, f1, dxc, dxn.reshape(B, NP, 1))

    return (energy.reshape(B, 1), forces3.reshape(B, NP))


# speedup vs baseline: 2.0905x; 1.0969x over previous
import functools

import jax
import jax.numpy as jnp
import numpy as np
from jax import lax
from jax.experimental import pallas as pl
from jax.experimental.pallas import tpu as pltpu
from jax.experimental.pallas import tpu_sc as plsc

B = 2
NP = 10000
MNN = 32
LBOX = 10.0
KF = 500
KP = 512
FFTC = 4
P = NP * MNN
BP = B * P
RT = 1000
TP = RT * MNN
NT = 10
RTP = 80
TPP = RTP * MNN
NTP = NP // RTP
NW = 32
PPW = BP // NW

_KV = (2.0 * np.pi / LBOX) * np.arange(-(KF // 2), KF // 2, dtype=np.float32)
KPAD = np.zeros((1, KP), dtype=np.float32)
KPAD[0, :KF] = _KV

_BF = jnp.bfloat16


def _q(a):
    return a.astype(_BF)


def _dotq(a, b):
    return jnp.dot(_q(a), _q(b), preferred_element_type=jnp.float32)


def _full_spec(shape):
    nd = len(shape)
    return pl.BlockSpec(shape, lambda b, t: (0,) * nd)


def _wb(wrefs, i):
    return wrefs[2 * i][...], wrefs[2 * i + 1][...]


def _dup(x, d):
    return jnp.concatenate([x[:, :d], x[:, :d], x[:, d:], x[:, d:]], axis=-1)


def _pair_mlp_fwd(g1, g0, wrefs):
    w0, b0 = _wb(wrefs, 0)
    xd = jnp.concatenate([g1, g1, g0, g0], axis=-1)
    y = jnp.tanh(xd * w0 + b0)
    ys = [y]
    x = y + jnp.concatenate([g1, g1, g0, g0], axis=-1)
    for i in range(1, 5):
        W, b = _wb(wrefs, i)
        y = jnp.tanh(_dotq(x, W) + b)
        ys.append(y)
        x = y + _dup(x, W.shape[0] // 2)
    return x, ys


def _pair_mlp_bwd(ys, wrefs, dx):
    for i in reversed(range(1, 5)):
        W, _ = _wb(wrefs, i)
        y = ys[i]
        dpre = dx * (1.0 - y * y)
        din = W.shape[0] // 2
        bw = _dotq(dpre, W.T)
        dx = jnp.concatenate(
            [bw[:, :din] + dx[:, :din] + dx[:, din:2 * din],
             bw[:, din:] + dx[:, 2 * din:3 * din] + dx[:, 3 * din:]], axis=-1)
    w0, _ = _wb(wrefs, 0)
    y = ys[0]
    dpre = dx * (1.0 - y * y)
    dp = _q(dpre).astype(jnp.float32) * _q(w0).astype(jnp.float32)
    dg1 = (dp[:, 0:1] + dp[:, 1:2]) + dx[:, 0:1] + dx[:, 1:2]
    dg0 = (dp[:, 2:3] + dp[:, 3:4]) + dx[:, 2:3] + dx[:, 3:4]
    return dg1, dg0


def _pair_geom(d, vd):
    dist = jnp.abs(d)
    ok = vd > 0.0
    safe = jnp.where(ok, jnp.maximum(dist, 1e-6), 1.0)
    g0 = jnp.where(ok, 1.0 / safe, 0.0)
    g1 = jnp.where(ok, safe, 0.0)
    return dist, ok, safe, g0, g1


def _rho_body(x_ref, k_ref, dep_ref, rho_ref):
    del dep_ref
    ph = x_ref[0] * k_ref[...]
    c = jnp.cos(ph)
    s = jnp.sin(ph)
    blk = jnp.concatenate(
        [jnp.sum(c, axis=0, keepdims=True), -jnp.sum(s, axis=0, keepdims=True)],
        axis=0)

    @pl.when(pl.program_id(1) == 0)
    def _():
        rho_ref[0] = blk

    @pl.when(pl.program_id(1) != 0)
    def _():
        rho_ref[0] = rho_ref[0] + blk


def _pairfwd_body(diff_ref, valid_ref, *refs):
    wrefs, d_out = refs[:10], refs[10]
    d = diff_ref[0]
    vd = valid_ref[0]
    _, _, _, g0, g1 = _pair_geom(d, vd)
    pp, _ = _pair_mlp_fwd(g1, g0, wrefs)
    ll = pp * g0
    d_out[0] = jnp.sum(ll.reshape(RTP, MNN, 64), axis=1)


def _lr_body(x_ref, k_ref, rho_ref, mult_ref, lr_ref):
    ph = x_ref[0] * k_ref[...]
    c = jnp.cos(ph)
    s = jnp.sin(ph)
    are = mult_ref[...] * rho_ref[0, 0:1, :]
    aim = mult_ref[...] * rho_ref[0, 1:2, :]
    f = (jnp.einsum('tk,ck->tc', _q(c), _q(are),
                    preferred_element_type=jnp.float32)
         - jnp.einsum('tk,ck->tc', _q(s), _q(aim),
                      preferred_element_type=jnp.float32)) / NP
    lr_ref[0] = f


def _head_body(d_ref, lr_ref, *refs):
    wlr = refs[:10]
    wfit = refs[10:20]
    linw_ref, linb_ref, e_ref, dd_ref, dlr_ref = refs[20:]
    lacts = []
    x = lr_ref[0]
    for i in range(5):
        W, b = _wb(wlr, i)
        pre = _dotq(x, W) + b
        y = jnp.maximum(pre, 0.0)
        lacts.append(y)
        x = y if i == 0 else y + jnp.concatenate([x, x], axis=-1)
    dlong = jnp.concatenate([d_ref[0], x], axis=-1)
    facts = []
    x = dlong
    for i in range(5):
        W, b = _wb(wfit, i)
        y = jnp.tanh(_dotq(x, W) + b)
        facts.append((y, W))
        x = y
    lw = linw_ref[...]
    f = x * lw[0][None, :] + linb_ref[...]
    esum = jnp.sum(f).reshape(1, 1, 1)

    @pl.when(pl.program_id(1) == 0)
    def _():
        e_ref[...] = esum

    @pl.when(pl.program_id(1) != 0)
    def _():
        e_ref[...] = e_ref[...] + esum

    dx = jnp.broadcast_to(lw[0:1, 0:1], (RT, 1))
    for (y, W) in reversed(facts):
        dpre = dx * (1.0 - y * y)
        if W.shape[1] == 1:
            dx = dpre * W[:, 0][None, :]
        else:
            dx = _dotq(dpre, W.T)
    dd_ref[0] = dx[:, :64]
    dx = dx[:, 64:]
    for i in reversed(range(5)):
        W, _ = _wb(wlr, i)
        y = lacts[i]
        dpre = dx * (y > 0.0)
        dxin = _dotq(dpre, W.T)
        if i > 0:
            din = W.shape[0]
            dxin = dxin + dx[:, :din] + dx[:, din:]
        dx = dxin
    dlr_ref[0] = dx


def _pairbwd_body(diff_ref, valid_ref, dd_ref, *refs):
    wrefs, gd_ref, dxc_ref = refs[:10], refs[10], refs[11]
    d = diff_ref[0]
    vd = valid_ref[0]
    dist, ok, safe, g0, g1 = _pair_geom(d, vd)
    pp, ys = _pair_mlp_fwd(g1, g0, wrefs)
    dd = dd_ref[0]
    uv = jnp.broadcast_to(dd[:, None, :], (RTP, MNN, 64)).reshape(TPP, 64)
    dg1, dg0p = _pair_mlp_bwd(ys, wrefs, uv * g0)
    dg0 = jnp.sum(uv * pp, axis=1, keepdims=True) + dg0p
    ddist = jnp.where(ok & (dist > 1e-6), dg1 - dg0 / (safe * safe), 0.0)
    gd = ddist * jnp.sign(d)
    gd_ref[0] = gd
    dxc_ref[0] = -jnp.sum(gd.reshape(RTP, MNN, 1), axis=1)


def _nufftbwd1_body(x_ref, dlr_ref, mult_ref, rho_ref, k_ref, dep_ref,
                    da_ref, f1_ref):
    del dep_ref
    ph = x_ref[0] * k_ref[...]
    c = jnp.cos(ph)
    s = jnp.sin(ph)
    gn = dlr_ref[0] * (1.0 / NP)
    gq = _q(gn)
    are = mult_ref[...] * rho_ref[0, 0:1, :]
    aim = mult_ref[...] * rho_ref[0, 1:2, :]
    dc_f = jnp.einsum('tc,ck->tk', gq, _q(are),
                      preferred_element_type=jnp.float32)
    ds_f = -jnp.einsum('tc,ck->tk', gq, _q(aim),
                       preferred_element_type=jnp.float32)
    kv = k_ref[...]
    f1_ref[0] = jnp.sum((-s * dc_f + c * ds_f) * kv, axis=1, keepdims=True)
    dat = jnp.einsum('tc,tk->ck', gq, _q(c), preferred_element_type=jnp.float32)
    dbt = -jnp.einsum('tc,tk->ck', gq, _q(s), preferred_element_type=jnp.float32)
    blk = jnp.concatenate([dat, dbt], axis=0)

    @pl.when(pl.program_id(1) == 0)
    def _():
        da_ref[0] = blk

    @pl.when(pl.program_id(1) != 0)
    def _():
        da_ref[0] = da_ref[0] + blk


def _combine_body(part_ref, dxn_ref):
    dxn_ref[0] = jnp.sum(part_ref[...], axis=0, keepdims=True)


def _force_body(x_ref, da_ref, mult_ref, k_ref, f1_ref, dxc_ref, dxn_ref,
                out_ref):
    ph = x_ref[0] * k_ref[...]
    c = jnp.cos(ph)
    s = jnp.sin(ph)
    kv = k_ref[...]
    m = mult_ref[...]
    drr = jnp.sum(da_ref[0, :FFTC, :] * m, axis=0, keepdims=True)
    dri = jnp.sum(da_ref[0, FFTC:, :] * m, axis=0, keepdims=True)
    t2 = jnp.sum((-s * drr - c * dri) * kv, axis=1, keepdims=True)
    out_ref[0] = -(f1_ref[0] + t2 + dxc_ref[0] + dxn_ref[0])


def _sc_gather(x, neigh_flat):
    mesh = plsc.VectorSubcoreMesh(core_axis_name="c", subcore_axis_name="s")

    @functools.partial(
        pl.kernel, mesh=mesh,
        out_type=[jax.ShapeDtypeStruct((BP,), jnp.float32),
                  jax.ShapeDtypeStruct((BP,), jnp.float32)],
        scratch_types=[pltpu.VMEM((NP,), jnp.float32),
                       pltpu.VMEM((PPW,), jnp.int32),
                       pltpu.VMEM((PPW,), jnp.float32),
                       pltpu.VMEM((PPW,), jnp.float32)],
        compiler_params=pltpu.CompilerParams(needs_layout_passes=False,
                                             has_side_effects=True),
    )
    def k(x_hbm, n_hbm, diff_hbm, valid_hbm, xtab, nbuf, dbuf, vbuf):
        wid = lax.axis_index("s") * 2 + lax.axis_index("c")
        b = wid // 16
        base = wid * PPW
        pltpu.sync_copy(x_hbm.at[b], xtab)
        pltpu.sync_copy(n_hbm.at[pl.ds(base, PPW)], nbuf)
        woff = (wid % 16) * PPW

        def body(j, _):
            o = j * 16
            nv = nbuf[pl.ds(o, 16)]
            pb = woff + o + lax.broadcasted_iota(jnp.int32, (16,), 0)
            iv = lax.shift_right_logical(pb, 5)
            xj = plsc.load_gather(xtab, [nv])
            xi = plsc.load_gather(xtab, [iv])
            d0 = xj - xi
            tt = d0 * jnp.float32(1.0 / LBOX)
            wrap = jnp.where(jnp.abs(tt) > 0.5, jnp.sign(tt), 0.0)
            dbuf[pl.ds(o, 16)] = d0 - jnp.float32(LBOX) * wrap
            okv = (nv >= 0) & (nv != iv)
            vbuf[pl.ds(o, 16)] = jnp.where(okv, 1.0, 0.0).astype(jnp.float32)
            return 0

        lax.fori_loop(0, PPW // 16, body, 0)
        pltpu.sync_copy(dbuf, diff_hbm.at[pl.ds(base, PPW)])
        pltpu.sync_copy(vbuf, valid_hbm.at[pl.ds(base, PPW)])

    return k(x, neigh_flat)


def _sc_scatter(gd_flat, neigh_flat):
    mesh = plsc.VectorSubcoreMesh(core_axis_name="c", subcore_axis_name="s")

    @functools.partial(
        pl.kernel, mesh=mesh,
        out_type=jax.ShapeDtypeStruct((NW, NP), jnp.float32),
        scratch_types=[pltpu.VMEM((NP,), jnp.float32),
                       pltpu.VMEM((PPW,), jnp.int32),
                       pltpu.VMEM((PPW,), jnp.float32)],
        compiler_params=pltpu.CompilerParams(needs_layout_passes=False,
                                             has_side_effects=True),
    )
    def k(g_hbm, n_hbm, part_hbm, acc, nbuf, gbuf):
        wid = lax.axis_index("s") * 2 + lax.axis_index("c")
        base = wid * PPW
        pltpu.sync_copy(n_hbm.at[pl.ds(base, PPW)], nbuf)
        pltpu.sync_copy(g_hbm.at[pl.ds(base, PPW)], gbuf)

        def zbody(j, _):
            acc[pl.ds(j * 16, 16)] = jnp.zeros((16,), jnp.float32)
            return 0

        lax.fori_loop(0, NP // 16, zbody, 0)

        def body(j, _):
            o = j * 16
            plsc.addupdate_scatter(acc, [nbuf[pl.ds(o, 16)]], gbuf[pl.ds(o, 16)])
            return 0

        lax.fori_loop(0, PPW // 16, body, 0)
        pltpu.sync_copy(acc, part_hbm.at[wid])

    return k(gd_flat, neigh_flat)


def kernel(inputs, neighList, params):
    x3 = inputs[:, :, None]
    neigh_flat = neighList.reshape(BP)
    kpad = jnp.asarray(KPAD)
    multp = jnp.concatenate(
        [params['mult'], jnp.zeros((FFTC, KP - KF), jnp.float32)], axis=1)

    def wlist(ps):
        out = []
        for (W, b) in ps:
            out.append(W)
            out.append(b.reshape(1, -1))
        return out

    def pairw(p1s, p2s):
        out = []
        for i, ((W1, b1), (W2, b2)) in enumerate(zip(p1s, p2s)):
            if i == 0:
                Wc = jnp.concatenate([W1, W2], axis=1)
            else:
                z = jnp.zeros_like(W1)
                Wc = jnp.concatenate(
                    [jnp.concatenate([W1, z], axis=1),
                     jnp.concatenate([z, W2], axis=1)], axis=0)
            out.append(Wc)
            out.append(jnp.concatenate([b1, b2]).reshape(1, -1))
        return out

    pair_w = pairw(params['pyr'], params['pyrInv'])
    pyrlr_w = wlist(params['pyrLR'])
    fit_w = wlist(params['fit'])
    linw, linb = params['lin']
    linb = linb.reshape(1, 1)

    w_specs = [_full_spec(w.shape) for w in pair_w]
    grid = (B, NT)

    diff_flat, valid_flat = _sc_gather(inputs, neigh_flat)
    diff3 = diff_flat.reshape(B, P, 1)
    valid3 = valid_flat.reshape(B, P, 1)

    rho = pl.pallas_call(
        _rho_body, grid=grid,
        in_specs=[pl.BlockSpec((1, RT, 1), lambda b, t: (b, t, 0)),
                  _full_spec((1, KP)),
                  pl.BlockSpec((1, 8, 1), lambda b, t: (0, 0, 0))],
        out_specs=pl.BlockSpec((1, 2, KP), lambda b, t: (b, 0, 0)),
        out_shape=jax.ShapeDtypeStruct((B, 2, KP), jnp.float32),
    )(x3, kpad, diff3)

    D = pl.pallas_call(
        _pairfwd_body, grid=(B, NTP),
        in_specs=[pl.BlockSpec((1, TPP, 1), lambda b, t: (b, t, 0)),
                  pl.BlockSpec((1, TPP, 1), lambda b, t: (b, t, 0))] + w_specs,
        out_specs=pl.BlockSpec((1, RTP, 64), lambda b, t: (b, t, 0)),
        out_shape=jax.ShapeDtypeStruct((B, NP, 64), jnp.float32),
    )(diff3, valid3, *pair_w)

    lr = pl.pallas_call(
        _lr_body, grid=grid,
        in_specs=[pl.BlockSpec((1, RT, 1), lambda b, t: (b, t, 0)),
                  _full_spec((1, KP)),
                  pl.BlockSpec((1, 2, KP), lambda b, t: (b, 0, 0)),
                  _full_spec((FFTC, KP))],
        out_specs=pl.BlockSpec((1, RT, FFTC), lambda b, t: (b, t, 0)),
        out_shape=jax.ShapeDtypeStruct((B, NP, FFTC), jnp.float32),
    )(x3, kpad, rho, multp)

    head_w = pyrlr_w + fit_w
    energy, dD, dlr = pl.pallas_call(
        _head_body, grid=grid,
        in_specs=[pl.BlockSpec((1, RT, 64), lambda b, t: (b, t, 0)),
                  pl.BlockSpec((1, RT, FFTC), lambda b, t: (b, t, 0))]
                 + [_full_spec(w.shape) for w in head_w]
                 + [_full_spec((1, 1)), _full_spec((1, 1))],
        out_specs=[pl.BlockSpec((1, 1, 1), lambda b, t: (b, 0, 0)),
                   pl.BlockSpec((1, RT, 64), lambda b, t: (b, t, 0)),
                   pl.BlockSpec((1, RT, FFTC), lambda b, t: (b, t, 0))],
        out_shape=[jax.ShapeDtypeStruct((B, 1, 1), jnp.float32),
                   jax.ShapeDtypeStruct((B, NP, 64), jnp.float32),
                   jax.ShapeDtypeStruct((B, NP, FFTC), jnp.float32)],
    )(D, lr, *head_w, linw, linb)

    gd3, dxc = pl.pallas_call(
        _pairbwd_body, grid=(B, NTP),
        in_specs=[pl.BlockSpec((1, TPP, 1), lambda b, t: (b, t, 0)),
                  pl.BlockSpec((1, TPP, 1), lambda b, t: (b, t, 0)),
                  pl.BlockSpec((1, RTP, 64), lambda b, t: (b, t, 0))] + w_specs,
        out_specs=[pl.BlockSpec((1, TPP, 1), lambda b, t: (b, t, 0)),
                   pl.BlockSpec((1, RTP, 1), lambda b, t: (b, t, 0))],
        out_shape=[jax.ShapeDtypeStruct((B, P, 1), jnp.float32),
                   jax.ShapeDtypeStruct((B, NP, 1), jnp.float32)],
    )(diff3, valid3, dD, *pair_w)

    part = _sc_scatter(gd3.reshape(BP), neigh_flat)

    dxn = pl.pallas_call(
        _combine_body, grid=(B,),
        in_specs=[pl.BlockSpec((16, NP), lambda b: (b, 0))],
        out_specs=pl.BlockSpec((1, 1, NP), lambda b: (b, 0, 0)),
        out_shape=jax.ShapeDtypeStruct((B, 1, NP), jnp.float32),
    )(part)

    da, f1 = pl.pallas_call(
        _nufftbwd1_body, grid=grid,
        in_specs=[pl.BlockSpec((1, RT, 1), lambda b, t: (b, t, 0)),
                  pl.BlockSpec((1, RT, FFTC), lambda b, t: (b, t, 0)),
                  _full_spec((FFTC, KP)),
                  pl.BlockSpec((1, 2, KP), lambda b, t: (b, 0, 0)),
                  _full_spec((1, KP)),
                  pl.BlockSpec((1, 1, NP), lambda b, t: (0, 0, 0))],
        out_specs=[pl.BlockSpec((1, 2 * FFTC, KP), lambda b, t: (b, 0, 0)),
                   pl.BlockSpec((1, RT, 1), lambda b, t: (b, t, 0))],
        out_shape=[jax.ShapeDtypeStruct((B, 2 * FFTC, KP), jnp.float32),
                   jax.ShapeDtypeStruct((B, NP, 1), jnp.float32)],
    )(x3, dlr, multp, rho, kpad, dxn)

    forces3 = pl.pallas_call(
        _force_body, grid=grid,
        in_specs=[pl.BlockSpec((1, RT, 1), lambda b, t: (b, t, 0)),
                  pl.BlockSpec((1, 2 * FFTC, KP), lambda b, t: (b, 0, 0)),
                  _full_spec((FFTC, KP)),
                  _full_spec((1, KP)),
                  pl.BlockSpec((1, RT, 1), lambda b, t: (b, t, 0)),
                  pl.BlockSpec((1, RT, 1), lambda b, t: (b, t, 0)),
                  pl.BlockSpec((1, RT, 1), lambda b, t: (b, t, 0))],
        out_specs=pl.BlockSpec((1, RT, 1), lambda b, t: (b, t, 0)),
        out_shape=jax.ShapeDtypeStruct((B, NP, 1), jnp.float32),
    )(x3, da, multp, kpad, f1, dxc, dxn.reshape(B, NP, 1))

    return (energy.reshape(B, 1), forces3.reshape(B, NP))


# speedup vs baseline: 2.2586x; 1.0804x over previous
import functools

import jax
import jax.numpy as jnp
import numpy as np
from jax import lax
from jax.experimental import pallas as pl
from jax.experimental.pallas import tpu as pltpu
from jax.experimental.pallas import tpu_sc as plsc

B = 2
NP = 10000
MNN = 32
LBOX = 10.0
KF = 500
KP = 512
FFTC = 4
P = NP * MNN
BP = B * P
RT = 1000
TP = RT * MNN
NT = 10
RTP = 200
TPP = RTP * MNN
NTP = NP // RTP
NW = 32
PPW = BP // NW

_KV = (2.0 * np.pi / LBOX) * np.arange(-(KF // 2), KF // 2, dtype=np.float32)
KPAD = np.zeros((1, KP), dtype=np.float32)
KPAD[0, :KF] = _KV

_BF = jnp.bfloat16


def _q(a):
    return a.astype(_BF)


def _dotq(a, b):
    return jnp.dot(_q(a), _q(b), preferred_element_type=jnp.float32)


def _full_spec(shape):
    nd = len(shape)
    return pl.BlockSpec(shape, lambda b, t: (0,) * nd)


def _wb(wrefs, i):
    return wrefs[2 * i][...], wrefs[2 * i + 1][...]


def _dup(x, d):
    return jnp.concatenate([x[:, :d], x[:, :d], x[:, d:], x[:, d:]], axis=-1)


def _pair_mlp_fwd(g1, g0, wrefs):
    w0, b0 = _wb(wrefs, 0)
    xd = jnp.concatenate([g1, g1, g0, g0], axis=-1)
    y = jnp.tanh(xd * w0 + b0)
    ys = [y]
    x = y + jnp.concatenate([g1, g1, g0, g0], axis=-1)
    for i in range(1, 5):
        W, b = _wb(wrefs, i)
        y = jnp.tanh(_dotq(x, W) + b)
        ys.append(y)
        x = y + _dup(x, W.shape[0] // 2)
    return x, ys


def _pair_mlp_bwd(ys, wrefs, dx):
    for i in reversed(range(1, 5)):
        W, _ = _wb(wrefs, i)
        y = ys[i]
        dpre = dx * (1.0 - y * y)
        din = W.shape[0] // 2
        bw = _dotq(dpre, W.T)
        dx = jnp.concatenate(
            [bw[:, :din] + dx[:, :din] + dx[:, din:2 * din],
             bw[:, din:] + dx[:, 2 * din:3 * din] + dx[:, 3 * din:]], axis=-1)
    w0, _ = _wb(wrefs, 0)
    y = ys[0]
    dpre = dx * (1.0 - y * y)
    dp = _q(dpre).astype(jnp.float32) * _q(w0).astype(jnp.float32)
    dg1 = (dp[:, 0:1] + dp[:, 1:2]) + dx[:, 0:1] + dx[:, 1:2]
    dg0 = (dp[:, 2:3] + dp[:, 3:4]) + dx[:, 2:3] + dx[:, 3:4]
    return dg1, dg0


def _pair_geom(d):
    dist = jnp.abs(d)
    ok = dist < 6.0
    safe = jnp.where(ok, jnp.maximum(dist, 1e-6), 1.0)
    g0 = jnp.where(ok, 1.0 / safe, 0.0)
    g1 = jnp.where(ok, safe, 0.0)
    return dist, ok, safe, g0, g1


def _rho_body(x_ref, k_ref, dep_ref, rho_ref):
    del dep_ref
    ph = x_ref[0] * k_ref[...]
    c = jnp.cos(ph)
    s = jnp.sin(ph)
    blk = jnp.concatenate(
        [jnp.sum(c, axis=0, keepdims=True), -jnp.sum(s, axis=0, keepdims=True)],
        axis=0)

    @pl.when(pl.program_id(1) == 0)
    def _():
        rho_ref[0] = blk

    @pl.when(pl.program_id(1) != 0)
    def _():
        rho_ref[0] = rho_ref[0] + blk


def _pairfwd_body(diff_ref, *refs):
    wrefs, d_out = refs[:10], refs[10]
    d = diff_ref[0]
    _, _, _, g0, g1 = _pair_geom(d)
    pp, _ = _pair_mlp_fwd(g1, g0, wrefs)
    ll = pp * g0
    d_out[0] = jnp.sum(ll.reshape(RTP, MNN, 64), axis=1)


def _lr_body(x_ref, k_ref, rho_ref, mult_ref, lr_ref):
    ph = x_ref[0] * k_ref[...]
    c = jnp.cos(ph)
    s = jnp.sin(ph)
    are = mult_ref[...] * rho_ref[0, 0:1, :]
    aim = mult_ref[...] * rho_ref[0, 1:2, :]
    f = (jnp.einsum('tk,ck->tc', _q(c), _q(are),
                    preferred_element_type=jnp.float32)
         - jnp.einsum('tk,ck->tc', _q(s), _q(aim),
                      preferred_element_type=jnp.float32)) / NP
    lr_ref[0] = f


def _head_body(d_ref, lr_ref, *refs):
    wlr = refs[:10]
    wfit = refs[10:20]
    linw_ref, linb_ref, e_ref, dd_ref, dlr_ref = refs[20:]
    lacts = []
    x = lr_ref[0]
    for i in range(5):
        W, b = _wb(wlr, i)
        pre = _dotq(x, W) + b
        y = jnp.maximum(pre, 0.0)
        lacts.append(y)
        x = y if i == 0 else y + jnp.concatenate([x, x], axis=-1)
    dlong = jnp.concatenate([d_ref[0], x], axis=-1)
    facts = []
    x = dlong
    for i in range(5):
        W, b = _wb(wfit, i)
        y = jnp.tanh(_dotq(x, W) + b)
        facts.append((y, W))
        x = y
    lw = linw_ref[...]
    f = x * lw[0][None, :] + linb_ref[...]
    esum = jnp.sum(f).reshape(1, 1, 1)

    @pl.when(pl.program_id(1) == 0)
    def _():
        e_ref[...] = esum

    @pl.when(pl.program_id(1) != 0)
    def _():
        e_ref[...] = e_ref[...] + esum

    dx = jnp.broadcast_to(lw[0:1, 0:1], (RT, 1))
    for (y, W) in reversed(facts):
        dpre = dx * (1.0 - y * y)
        if W.shape[1] == 1:
            dx = dpre * W[:, 0][None, :]
        else:
            dx = _dotq(dpre, W.T)
    dd_ref[0] = dx[:, :64]
    dx = dx[:, 64:]
    for i in reversed(range(5)):
        W, _ = _wb(wlr, i)
        y = lacts[i]
        dpre = dx * (y > 0.0)
        dxin = _dotq(dpre, W.T)
        if i > 0:
            din = W.shape[0]
            dxin = dxin + dx[:, :din] + dx[:, din:]
        dx = dxin
    dlr_ref[0] = dx


def _pairbwd_body(diff_ref, dd_ref, *refs):
    wrefs, gd_ref, dxc_ref = refs[:10], refs[10], refs[11]
    d = diff_ref[0]
    dist, ok, safe, g0, g1 = _pair_geom(d)
    pp, ys = _pair_mlp_fwd(g1, g0, wrefs)
    dd = dd_ref[0]
    uv = jnp.broadcast_to(dd[:, None, :], (RTP, MNN, 64)).reshape(TPP, 64)
    dg1, dg0p = _pair_mlp_bwd(ys, wrefs, uv * g0)
    dg0 = jnp.sum(uv * pp, axis=1, keepdims=True) + dg0p
    ddist = jnp.where(ok & (dist > 1e-6), dg1 - dg0 / (safe * safe), 0.0)
    gd = ddist * jnp.sign(d)
    gd_ref[0] = gd
    dxc_ref[0] = -jnp.sum(gd.reshape(RTP, MNN, 1), axis=1)


def _nufftbwd1_body(x_ref, dlr_ref, mult_ref, rho_ref, k_ref, dep_ref,
                    da_ref, f1_ref):
    del dep_ref
    ph = x_ref[0] * k_ref[...]
    c = jnp.cos(ph)
    s = jnp.sin(ph)
    gn = dlr_ref[0] * (1.0 / NP)
    gq = _q(gn)
    are = mult_ref[...] * rho_ref[0, 0:1, :]
    aim = mult_ref[...] * rho_ref[0, 1:2, :]
    dc_f = jnp.einsum('tc,ck->tk', gq, _q(are),
                      preferred_element_type=jnp.float32)
    ds_f = -jnp.einsum('tc,ck->tk', gq, _q(aim),
                       preferred_element_type=jnp.float32)
    kv = k_ref[...]
    f1_ref[0] = jnp.sum((-s * dc_f + c * ds_f) * kv, axis=1, keepdims=True)
    dat = jnp.einsum('tc,tk->ck', gq, _q(c), preferred_element_type=jnp.float32)
    dbt = -jnp.einsum('tc,tk->ck', gq, _q(s), preferred_element_type=jnp.float32)
    blk = jnp.concatenate([dat, dbt], axis=0)

    @pl.when(pl.program_id(1) == 0)
    def _():
        da_ref[0] = blk

    @pl.when(pl.program_id(1) != 0)
    def _():
        da_ref[0] = da_ref[0] + blk


def _combine_body(part_ref, dxn_ref):
    dxn_ref[0] = jnp.sum(part_ref[...], axis=0, keepdims=True)


def _force_body(x_ref, da_ref, mult_ref, k_ref, f1_ref, dxc_ref, dxn_ref,
                out_ref):
    ph = x_ref[0] * k_ref[...]
    c = jnp.cos(ph)
    s = jnp.sin(ph)
    kv = k_ref[...]
    m = mult_ref[...]
    drr = jnp.sum(da_ref[0, :FFTC, :] * m, axis=0, keepdims=True)
    dri = jnp.sum(da_ref[0, FFTC:, :] * m, axis=0, keepdims=True)
    t2 = jnp.sum((-s * drr - c * dri) * kv, axis=1, keepdims=True)
    out_ref[0] = -(f1_ref[0] + t2 + dxc_ref[0] + dxn_ref[0])


def _sc_gather(x, neigh_flat):
    mesh = plsc.VectorSubcoreMesh(core_axis_name="c", subcore_axis_name="s")

    @functools.partial(
        pl.kernel, mesh=mesh,
        out_type=jax.ShapeDtypeStruct((BP,), jnp.float32),
        scratch_types=[pltpu.VMEM((NP,), jnp.float32),
                       pltpu.VMEM((PPW,), jnp.int32),
                       pltpu.VMEM((PPW,), jnp.float32)],
        compiler_params=pltpu.CompilerParams(needs_layout_passes=False,
                                             has_side_effects=True),
    )
    def k(x_hbm, n_hbm, diff_hbm, xtab, nbuf, dbuf):
        wid = lax.axis_index("s") * 2 + lax.axis_index("c")
        b = wid // 16
        base = wid * PPW
        pltpu.sync_copy(x_hbm.at[b], xtab)
        pltpu.sync_copy(n_hbm.at[pl.ds(base, PPW)], nbuf)
        woff = (wid % 16) * PPW

        def body(j, _):
            o = j * 16
            nv = nbuf[pl.ds(o, 16)]
            pb = woff + o + lax.broadcasted_iota(jnp.int32, (16,), 0)
            iv = lax.shift_right_logical(pb, 5)
            xj = plsc.load_gather(xtab, [nv])
            xi = plsc.load_gather(xtab, [iv])
            d0 = xj - xi
            tt = d0 * jnp.float32(1.0 / LBOX)
            wrap = jnp.where(jnp.abs(tt) > 0.5, jnp.sign(tt), 0.0)
            okv = (nv >= 0) & (nv != iv)
            dw = d0 - jnp.float32(LBOX) * wrap
            dbuf[pl.ds(o, 16)] = jnp.where(okv, dw, jnp.float32(1e9))
            return 0

        lax.fori_loop(0, PPW // 16, body, 0)
        pltpu.sync_copy(dbuf, diff_hbm.at[pl.ds(base, PPW)])

    return k(x, neigh_flat)


def _sc_scatter(gd_flat, neigh_flat):
    mesh = plsc.VectorSubcoreMesh(core_axis_name="c", subcore_axis_name="s")

    @functools.partial(
        pl.kernel, mesh=mesh,
        out_type=jax.ShapeDtypeStruct((NW, NP), jnp.float32),
        scratch_types=[pltpu.VMEM((NP,), jnp.float32),
                       pltpu.VMEM((PPW,), jnp.int32),
                       pltpu.VMEM((PPW,), jnp.float32)],
        compiler_params=pltpu.CompilerParams(needs_layout_passes=False,
                                             has_side_effects=True),
    )
    def k(g_hbm, n_hbm, part_hbm, acc, nbuf, gbuf):
        wid = lax.axis_index("s") * 2 + lax.axis_index("c")
        base = wid * PPW
        pltpu.sync_copy(n_hbm.at[pl.ds(base, PPW)], nbuf)
        pltpu.sync_copy(g_hbm.at[pl.ds(base, PPW)], gbuf)

        def zbody(j, _):
            acc[pl.ds(j * 16, 16)] = jnp.zeros((16,), jnp.float32)
            return 0

        lax.fori_loop(0, NP // 16, zbody, 0)

        def body(j, _):
            o = j * 16
            plsc.addupdate_scatter(acc, [nbuf[pl.ds(o, 16)]], gbuf[pl.ds(o, 16)])
            return 0

        lax.fori_loop(0, PPW // 16, body, 0)
        pltpu.sync_copy(acc, part_hbm.at[wid])

    return k(gd_flat, neigh_flat)


def kernel(inputs, neighList, params):
    x3 = inputs[:, :, None]
    neigh_flat = neighList.reshape(BP)
    kpad = jnp.asarray(KPAD)
    multp = jnp.concatenate(
        [params['mult'], jnp.zeros((FFTC, KP - KF), jnp.float32)], axis=1)

    def wlist(ps):
        out = []
        for (W, b) in ps:
            out.append(W)
            out.append(b.reshape(1, -1))
        return out

    def pairw(p1s, p2s):
        out = []
        for i, ((W1, b1), (W2, b2)) in enumerate(zip(p1s, p2s)):
            if i == 0:
                Wc = jnp.concatenate([W1, W2], axis=1)
            else:
                z = jnp.zeros_like(W1)
                Wc = jnp.concatenate(
                    [jnp.concatenate([W1, z], axis=1),
                     jnp.concatenate([z, W2], axis=1)], axis=0)
            out.append(Wc)
            out.append(jnp.concatenate([b1, b2]).reshape(1, -1))
        return out

    pair_w = pairw(params['pyr'], params['pyrInv'])
    pyrlr_w = wlist(params['pyrLR'])
    fit_w = wlist(params['fit'])
    linw, linb = params['lin']
    linb = linb.reshape(1, 1)

    w_specs = [_full_spec(w.shape) for w in pair_w]
    grid = (B, NT)

    diff_flat = _sc_gather(inputs, neigh_flat)
    diff3 = diff_flat.reshape(B, P, 1)

    rho = pl.pallas_call(
        _rho_body, grid=grid,
        in_specs=[pl.BlockSpec((1, RT, 1), lambda b, t: (b, t, 0)),
                  _full_spec((1, KP)),
                  pl.BlockSpec((1, 8, 1), lambda b, t: (0, 0, 0))],
        out_specs=pl.BlockSpec((1, 2, KP), lambda b, t: (b, 0, 0)),
        out_shape=jax.ShapeDtypeStruct((B, 2, KP), jnp.float32),
    )(x3, kpad, diff3)

    D = pl.pallas_call(
        _pairfwd_body, grid=(B, NTP),
        in_specs=[pl.BlockSpec((1, TPP, 1), lambda b, t: (b, t, 0))] + w_specs,
        out_specs=pl.BlockSpec((1, RTP, 64), lambda b, t: (b, t, 0)),
        out_shape=jax.ShapeDtypeStruct((B, NP, 64), jnp.float32),
    )(diff3, *pair_w)

    lr = pl.pallas_call(
        _lr_body, grid=grid,
        in_specs=[pl.BlockSpec((1, RT, 1), lambda b, t: (b, t, 0)),
                  _full_spec((1, KP)),
                  pl.BlockSpec((1, 2, KP), lambda b, t: (b, 0, 0)),
                  _full_spec((FFTC, KP))],
        out_specs=pl.BlockSpec((1, RT, FFTC), lambda b, t: (b, t, 0)),
        out_shape=jax.ShapeDtypeStruct((B, NP, FFTC), jnp.float32),
    )(x3, kpad, rho, multp)

    head_w = pyrlr_w + fit_w
    energy, dD, dlr = pl.pallas_call(
        _head_body, grid=grid,
        in_specs=[pl.BlockSpec((1, RT, 64), lambda b, t: (b, t, 0)),
                  pl.BlockSpec((1, RT, FFTC), lambda b, t: (b, t, 0))]
                 + [_full_spec(w.shape) for w in head_w]
                 + [_full_spec((1, 1)), _full_spec((1, 1))],
        out_specs=[pl.BlockSpec((1, 1, 1), lambda b, t: (b, 0, 0)),
                   pl.BlockSpec((1, RT, 64), lambda b, t: (b, t, 0)),
                   pl.BlockSpec((1, RT, FFTC), lambda b, t: (b, t, 0))],
        out_shape=[jax.ShapeDtypeStruct((B, 1, 1), jnp.float32),
                   jax.ShapeDtypeStruct((B, NP, 64), jnp.float32),
                   jax.ShapeDtypeStruct((B, NP, FFTC), jnp.float32)],
    )(D, lr, *head_w, linw, linb)

    gd3, dxc = pl.pallas_call(
        _pairbwd_body, grid=(B, NTP),
        in_specs=[pl.BlockSpec((1, TPP, 1), lambda b, t: (b, t, 0)),
                  pl.BlockSpec((1, RTP, 64), lambda b, t: (b, t, 0))] + w_specs,
        out_specs=[pl.BlockSpec((1, TPP, 1), lambda b, t: (b, t, 0)),
                   pl.BlockSpec((1, RTP, 1), lambda b, t: (b, t, 0))],
        out_shape=[jax.ShapeDtypeStruct((B, P, 1), jnp.float32),
                   jax.ShapeDtypeStruct((B, NP, 1), jnp.float32)],
    )(diff3, dD, *pair_w)

    part = _sc_scatter(gd3.reshape(BP), neigh_flat)

    dxn = pl.pallas_call(
        _combine_body, grid=(B,),
        in_specs=[pl.BlockSpec((16, NP), lambda b: (b, 0))],
        out_specs=pl.BlockSpec((1, 1, NP), lambda b: (b, 0, 0)),
        out_shape=jax.ShapeDtypeStruct((B, 1, NP), jnp.float32),
    )(part)

    da, f1 = pl.pallas_call(
        _nufftbwd1_body, grid=grid,
        in_specs=[pl.BlockSpec((1, RT, 1), lambda b, t: (b, t, 0)),
                  pl.BlockSpec((1, RT, FFTC), lambda b, t: (b, t, 0)),
                  _full_spec((FFTC, KP)),
                  pl.BlockSpec((1, 2, KP), lambda b, t: (b, 0, 0)),
                  _full_spec((1, KP)),
                  pl.BlockSpec((1, 1, NP), lambda b, t: (0, 0, 0))],
        out_specs=[pl.BlockSpec((1, 2 * FFTC, KP), lambda b, t: (b, 0, 0)),
                   pl.BlockSpec((1, RT, 1), lambda b, t: (b, t, 0))],
        out_shape=[jax.ShapeDtypeStruct((B, 2 * FFTC, KP), jnp.float32),
                   jax.ShapeDtypeStruct((B, NP, 1), jnp.float32)],
    )(x3, dlr, multp, rho, kpad, dxn)

    forces3 = pl.pallas_call(
        _force_body, grid=grid,
        in_specs=[pl.BlockSpec((1, RT, 1), lambda b, t: (b, t, 0)),
                  pl.BlockSpec((1, 2 * FFTC, KP), lambda b, t: (b, 0, 0)),
                  _full_spec((FFTC, KP)),
                  _full_spec((1, KP)),
                  pl.BlockSpec((1, RT, 1), lambda b, t: (b, t, 0)),
                  pl.BlockSpec((1, RT, 1), lambda b, t: (b, t, 0)),
                  pl.BlockSpec((1, RT, 1), lambda b, t: (b, t, 0))],
        out_specs=pl.BlockSpec((1, RT, 1), lambda b, t: (b, t, 0)),
        out_shape=jax.ShapeDtypeStruct((B, NP, 1), jnp.float32),
    )(x3, da, multp, kpad, f1, dxc, dxn.reshape(B, NP, 1))

    return (energy.reshape(B, 1), forces3.reshape(B, NP))
